# baseline (speedup 1.0000x reference)
import functools

import jax
import jax.numpy as jnp
from jax import lax
from jax.experimental import pallas as pl
from jax.experimental.pallas import tpu as pltpu
from jax.experimental.pallas import tpu_sc as plsc


@functools.lru_cache(maxsize=None)
def _build_gather(n: int, per_w: int, d: int, dp: int, nc: int, ns: int):
    chunk = 128
    nchunk = per_w // chunk
    ng = 5
    nt = 2
    super_steps = 2 * ng
    nsuper = (nchunk - ng) // super_steps
    assert nchunk == ng + nsuper * super_steps
    mesh = plsc.VectorSubcoreMesh(core_axis_name="c", subcore_axis_name="s")

    scratch = [pltpu.VMEM((per_w,), jnp.int32)]
    scratch += [pltpu.VMEM((chunk, dp), jnp.float32) for _ in range(ng)]
    scratch += [pltpu.VMEM((d, chunk), jnp.float32) for _ in range(nt)]
    scratch += [pltpu.SemaphoreType.DMA for _ in range(ng + nt)]

    @functools.partial(
        pl.kernel,
        mesh=mesh,
        out_type=jax.ShapeDtypeStruct((d, n), jnp.float32),
        scratch_types=scratch,
        compiler_params=pltpu.CompilerParams(
            use_tc_tiling_on_sc=True, disable_bounds_checks=True,
            needs_layout_passes=False),
    )
    def gather_kernel(x_hbm, table_hbm, out_hbm, idx_v, *rest):
        gbufs = rest[:ng]
        tbufs = rest[ng:ng + nt]
        gsem = rest[ng + nt:2 * ng + nt]
        wsem = rest[2 * ng + nt:2 * ng + 2 * nt]
        wid = lax.axis_index("s") * nc + lax.axis_index("c")
        base = wid * per_w
        pltpu.sync_copy(x_hbm.at[pl.ds(base, per_w)], idx_v)

        def valid(j):
            return base + j * chunk < n

        def gather_desc(j, gb):
            return pltpu.make_async_copy(
                table_hbm.at[idx_v.at[pl.ds(j * chunk, chunk)]],
                gbufs[gb], gsem[gb])

        def write_desc(j, tb):
            return pltpu.make_async_copy(
                tbufs[tb],
                out_hbm.at[:, pl.ds(base + j * chunk, chunk)],
                wsem[tb])

        def when_valid(j, fn):
            pl.when(valid(j))(fn)

        lane = lax.iota(jnp.int32, 16)

        def transpose_chunk(gb, tb):
            gbuf, tbuf = gbufs[gb], tbufs[tb]

            def body(k, carry):
                rotk = jnp.where(lane + k >= 16, lane + k - 16, lane + k)
                for ob in range(chunk // 16):
                    rows = lane + ob * 16
                    for fb in range(d // 16):
                        cols = rotk + fb * 16
                        v = plsc.load_gather(gbuf, [rows, cols])
                        plsc.store_scatter(tbuf, [cols, rows], v)
                return carry

            lax.fori_loop(0, 16, body, 0)

        def step(j, gb, tb, static_head):
            if static_head:
                gather_desc(j, gb).wait()
                if j >= nt:
                    when_valid(j - nt, write_desc(j - nt, tb).wait)
            else:
                when_valid(j, gather_desc(j, gb).wait)
                when_valid(j - nt, write_desc(j - nt, tb).wait)
            when_valid(j, lambda: transpose_chunk(gb, tb))
            when_valid(j, write_desc(j, tb).start)
            jj = j + ng
            pl.when((jj < nchunk) & valid(jj))(gather_desc(jj, gb).start)

        for b in range(ng):
            gather_desc(b, b).start()
        for b in range(ng):
            step(b, b, b % nt, True)

        def super_group(sg, carry):
            j0 = ng + super_steps * sg
            for bb in range(super_steps):
                step(j0 + bb, (ng + bb) % ng, (ng + bb) % nt, False)
            return carry

        lax.fori_loop(0, nsuper, super_group, 0)

        for j in range(nchunk - nt, nchunk):
            when_valid(j, write_desc(j, j % nt).wait)

    return gather_kernel


def kernel(x, table):
    n = x.shape[0]
    vocab, d = table.shape
    dp = 128
    info = plsc.get_sparse_core_info()
    nc, ns = info.num_cores, info.num_subcores
    nw = nc * ns
    chunk = 128
    per_w = -(-n // (nw * chunk)) * chunk
    n_pad = per_w * nw

    xi = x.astype(jnp.int32)
    pad = jnp.arange(n_pad - n, dtype=jnp.int32) % vocab
    x_pad = jnp.concatenate([xi, pad])
    table_p = jnp.pad(table, ((0, 0), (0, dp - d)))
    out_t = _build_gather(n, per_w, d, dp, nc, ns)(x_pad, table_p)
    return out_t.T


# speedup vs baseline: 1.3058x; 1.3058x over previous
import functools

import jax
import jax.numpy as jnp
from jax import lax
from jax.experimental import pallas as pl
from jax.experimental.pallas import tpu as pltpu
from jax.experimental.pallas import tpu_sc as plsc


@functools.lru_cache(maxsize=None)
def _build_gather(n: int, per_w: int, d: int, dp: int, nc: int, ns: int):
    chunk = 128
    nchunk = per_w // chunk
    ng = 5
    nt = 2
    super_steps = 2 * ng
    nsuper = (nchunk - ng) // super_steps
    assert nchunk == ng + nsuper * super_steps
    mesh = plsc.VectorSubcoreMesh(core_axis_name="c", subcore_axis_name="s")

    scratch = [pltpu.VMEM((per_w,), jnp.int32)]
    scratch += [pltpu.VMEM((chunk, dp), jnp.float32) for _ in range(ng)]
    scratch += [pltpu.VMEM((d, chunk), jnp.float32) for _ in range(nt)]
    scratch += [pltpu.SemaphoreType.DMA for _ in range(ng + nt)]

    @functools.partial(
        pl.kernel,
        mesh=mesh,
        out_type=jax.ShapeDtypeStruct((d, n), jnp.float32),
        scratch_types=scratch,
        compiler_params=pltpu.CompilerParams(
            use_tc_tiling_on_sc=True, disable_bounds_checks=True,
            needs_layout_passes=False),
    )
    def gather_kernel(x_hbm, table_hbm, out_hbm, idx_v, *rest):
        gbufs = rest[:ng]
        tbufs = rest[ng:ng + nt]
        gsem = rest[ng + nt:2 * ng + nt]
        wsem = rest[2 * ng + nt:2 * ng + 2 * nt]
        wid = lax.axis_index("s") * nc + lax.axis_index("c")
        base = wid * per_w
        pltpu.sync_copy(x_hbm.at[pl.ds(base, per_w)], idx_v)

        def valid(j):
            return base + j * chunk < n

        def gather_desc(j, gb):
            return pltpu.make_async_copy(
                table_hbm.at[idx_v.at[pl.ds(j * chunk, chunk)]],
                gbufs[gb], gsem[gb])

        def write_desc(j, tb):
            return pltpu.make_async_copy(
                tbufs[tb],
                out_hbm.at[:, pl.ds(base + j * chunk, chunk)],
                wsem[tb])

        def when_valid(j, fn):
            pl.when(valid(j))(fn)

        lane = lax.iota(jnp.int32, 16)
        rot = [jnp.where(lane + k >= 16, lane + k - 16, lane + k)
               for k in range(16)]

        def transpose_chunk(gb, tb):
            gbuf, tbuf = gbufs[gb], tbufs[tb]

            nfb = d // 16

            def body(i, carry):
                ob = i // nfb
                fb = i - ob * nfb
                rows = lane + ob * 16
                f0 = fb * 16
                cols = [rot[k] + f0 for k in range(16)]
                vs = [plsc.load_gather(gbuf, [rows, cols[k]])
                      for k in range(16)]
                for k in range(16):
                    plsc.store_scatter(tbuf, [cols[k], rows], vs[k])
                return carry

            lax.fori_loop(0, (chunk // 16) * nfb, body, 0)

        def step(j, gb, tb, static_head):
            if static_head:
                gather_desc(j, gb).wait()
                if j >= nt:
                    when_valid(j - nt, write_desc(j - nt, tb).wait)
            else:
                when_valid(j, gather_desc(j, gb).wait)
                when_valid(j - nt, write_desc(j - nt, tb).wait)
            when_valid(j, lambda: transpose_chunk(gb, tb))
            when_valid(j, write_desc(j, tb).start)
            jj = j + ng
            pl.when((jj < nchunk) & valid(jj))(gather_desc(jj, gb).start)

        for b in range(ng):
            gather_desc(b, b).start()
        for b in range(ng):
            step(b, b, b % nt, True)

        def super_group(sg, carry):
            j0 = ng + super_steps * sg
            for bb in range(super_steps):
                step(j0 + bb, (ng + bb) % ng, (ng + bb) % nt, False)
            return carry

        lax.fori_loop(0, nsuper, super_group, 0)

        for j in range(nchunk - nt, nchunk):
            when_valid(j, write_desc(j, j % nt).wait)

    return gather_kernel


def kernel(x, table):
    n = x.shape[0]
    vocab, d = table.shape
    dp = 128
    info = plsc.get_sparse_core_info()
    nc, ns = info.num_cores, info.num_subcores
    nw = nc * ns
    chunk = 128
    per_w = -(-n // (nw * chunk)) * chunk
    n_pad = per_w * nw

    xi = x.astype(jnp.int32)
    pad = jnp.arange(n_pad - n, dtype=jnp.int32) % vocab
    x_pad = jnp.concatenate([xi, pad])
    table_p = jnp.pad(table, ((0, 0), (0, dp - d)))
    out_t = _build_gather(n, per_w, d, dp, nc, ns)(x_pad, table_p)
    return out_t.T


# speedup vs baseline: 1.5289x; 1.1708x over previous
import functools

import jax
import jax.numpy as jnp
from jax import lax
from jax.experimental import pallas as pl
from jax.experimental.pallas import tpu as pltpu
from jax.experimental.pallas import tpu_sc as plsc


@functools.lru_cache(maxsize=None)
def _build_padfmt(vocab_p: int, d: int, dp: int, nc: int, ns: int):
    blk = 128
    nblk = vocab_p // blk
    nw = nc * ns
    slots = -(-nblk // nw)
    mesh = plsc.VectorSubcoreMesh(core_axis_name="c", subcore_axis_name="s")

    scratch = [pltpu.VMEM((d, blk), jnp.float32) for _ in range(2)]
    scratch += [pltpu.VMEM((blk, dp), jnp.float32) for _ in range(2)]
    scratch += [pltpu.SemaphoreType.DMA for _ in range(4)]

    @functools.partial(
        pl.kernel,
        mesh=mesh,
        out_type=jax.ShapeDtypeStruct((vocab_p, dp), jnp.float32),
        scratch_types=scratch,
        compiler_params=pltpu.CompilerParams(
            use_tc_tiling_on_sc=True, disable_bounds_checks=True,
            needs_layout_passes=False),
    )
    def padfmt_kernel(tt_hbm, out_hbm, *rest):
        sbufs = rest[:2]
        dbufs = rest[2:4]
        rsem = rest[4:6]
        wsem = rest[6:8]
        wid = lax.axis_index("s") * nc + lax.axis_index("c")
        base = wid * slots

        def valid(s):
            return base + s < nblk

        def read_desc(s):
            return pltpu.make_async_copy(
                tt_hbm.at[:, pl.ds((base + s) * blk, blk)],
                sbufs[s % 2], rsem[s % 2])

        def write_desc(s):
            return pltpu.make_async_copy(
                dbufs[s % 2],
                out_hbm.at[pl.ds((base + s) * blk, blk)],
                wsem[s % 2])

        def when_valid(s, fn):
            pl.when(valid(s))(fn)

        lane = lax.iota(jnp.int32, 16)
        rot = [jnp.where(lane + k >= 16, lane + k - 16, lane + k)
               for k in range(16)]

        def transpose_blk(b):
            sbuf, dbuf = sbufs[b], dbufs[b]
            nfb = d // 16

            def body(i, carry):
                vb = i // nfb
                fb = i - vb * nfb
                rows = lane + fb * 16
                v0 = vb * 16
                cols = [rot[k] + v0 for k in range(16)]
                vs = [plsc.load_gather(sbuf, [rows, cols[k]])
                      for k in range(16)]
                for k in range(16):
                    plsc.store_scatter(dbuf, [cols[k], rows], vs[k])
                return carry

            lax.fori_loop(0, (blk // 16) * nfb, body, 0)

        for s in range(min(2, slots)):
            when_valid(s, read_desc(s).start)
        for s in range(slots):
            when_valid(s, read_desc(s).wait)
            if s >= 2:
                when_valid(s - 2, write_desc(s - 2).wait)
            when_valid(s, lambda b=s % 2: transpose_blk(b))
            when_valid(s, write_desc(s).start)
            if s + 2 < slots:
                when_valid(s + 2, read_desc(s + 2).start)
        for s in range(max(0, slots - 2), slots):
            when_valid(s, write_desc(s).wait)

    return padfmt_kernel


@functools.lru_cache(maxsize=None)
def _build_gather(n: int, per_w: int, d: int, dp: int, nc: int, ns: int):
    chunk = 128
    nchunk = per_w // chunk
    ng = 5
    nt = 2
    super_steps = 2 * ng
    nsuper = (nchunk - ng) // super_steps
    assert nchunk == ng + nsuper * super_steps
    mesh = plsc.VectorSubcoreMesh(core_axis_name="c", subcore_axis_name="s")

    scratch = [pltpu.VMEM((per_w,), jnp.int32)]
    scratch += [pltpu.VMEM((chunk, dp), jnp.float32) for _ in range(ng)]
    scratch += [pltpu.VMEM((d, chunk), jnp.float32) for _ in range(nt)]
    scratch += [pltpu.SemaphoreType.DMA for _ in range(ng + nt)]

    @functools.partial(
        pl.kernel,
        mesh=mesh,
        out_type=jax.ShapeDtypeStruct((d, n), jnp.float32),
        scratch_types=scratch,
        compiler_params=pltpu.CompilerParams(
            use_tc_tiling_on_sc=True, disable_bounds_checks=True,
            needs_layout_passes=False),
    )
    def gather_kernel(x_hbm, table_hbm, out_hbm, idx_v, *rest):
        gbufs = rest[:ng]
        tbufs = rest[ng:ng + nt]
        gsem = rest[ng + nt:2 * ng + nt]
        wsem = rest[2 * ng + nt:2 * ng + 2 * nt]
        wid = lax.axis_index("s") * nc + lax.axis_index("c")
        base = wid * per_w
        pltpu.sync_copy(x_hbm.at[pl.ds(base, per_w)], idx_v)

        def valid(j):
            return base + j * chunk < n

        def gather_desc(j, gb):
            return pltpu.make_async_copy(
                table_hbm.at[idx_v.at[pl.ds(j * chunk, chunk)]],
                gbufs[gb], gsem[gb])

        def write_desc(j, tb):
            return pltpu.make_async_copy(
                tbufs[tb],
                out_hbm.at[:, pl.ds(base + j * chunk, chunk)],
                wsem[tb])

        def when_valid(j, fn):
            pl.when(valid(j))(fn)

        lane = lax.iota(jnp.int32, 16)
        rot = [jnp.where(lane + k >= 16, lane + k - 16, lane + k)
               for k in range(16)]

        def transpose_chunk(gb, tb):
            gbuf, tbuf = gbufs[gb], tbufs[tb]

            nfb = d // 16

            def body(i, carry):
                ob = i // nfb
                fb = i - ob * nfb
                rows = lane + ob * 16
                f0 = fb * 16
                cols = [rot[k] + f0 for k in range(16)]
                vs = [plsc.load_gather(gbuf, [rows, cols[k]])
                      for k in range(16)]
                for k in range(16):
                    plsc.store_scatter(tbuf, [cols[k], rows], vs[k])
                return carry

            lax.fori_loop(0, (chunk // 16) * nfb, body, 0)

        def step(j, gb, tb, static_head):
            if static_head:
                gather_desc(j, gb).wait()
                if j >= nt:
                    when_valid(j - nt, write_desc(j - nt, tb).wait)
            else:
                when_valid(j, gather_desc(j, gb).wait)
                when_valid(j - nt, write_desc(j - nt, tb).wait)
            when_valid(j, lambda: transpose_chunk(gb, tb))
            when_valid(j, write_desc(j, tb).start)
            jj = j + ng
            pl.when((jj < nchunk) & valid(jj))(gather_desc(jj, gb).start)

        for b in range(ng):
            gather_desc(b, b).start()
        for b in range(ng):
            step(b, b, b % nt, True)

        def super_group(sg, carry):
            j0 = ng + super_steps * sg
            for bb in range(super_steps):
                step(j0 + bb, (ng + bb) % ng, (ng + bb) % nt, False)
            return carry

        lax.fori_loop(0, nsuper, super_group, 0)

        for j in range(nchunk - nt, nchunk):
            when_valid(j, write_desc(j, j % nt).wait)

    return gather_kernel


def kernel(x, table):
    n = x.shape[0]
    vocab, d = table.shape
    dp = 128
    info = plsc.get_sparse_core_info()
    nc, ns = info.num_cores, info.num_subcores
    nw = nc * ns
    chunk = 128
    per_w = -(-n // (nw * chunk)) * chunk
    n_pad = per_w * nw

    xi = x.astype(jnp.int32)
    pad = jnp.arange(n_pad - n, dtype=jnp.int32) % vocab
    x_pad = jnp.concatenate([xi, pad])
    vocab_p = -(-vocab // 128) * 128
    table_p = _build_padfmt(vocab_p, d, dp, nc, ns)(table.T)
    out_t = _build_gather(n, per_w, d, dp, nc, ns)(x_pad, table_p)
    return out_t.T


# speedup vs baseline: 1.5340x; 1.0033x over previous
import functools

import jax
import jax.numpy as jnp
from jax import lax
from jax.experimental import pallas as pl
from jax.experimental.pallas import tpu as pltpu
from jax.experimental.pallas import tpu_sc as plsc


@functools.lru_cache(maxsize=None)
def _build_padfmt(vocab_p: int, d: int, dp: int, nc: int, ns: int):
    blk = 128
    nblk = vocab_p // blk
    nw = nc * ns
    slots = -(-nblk // nw)
    mesh = plsc.VectorSubcoreMesh(core_axis_name="c", subcore_axis_name="s")

    nr = 3
    scratch = [pltpu.VMEM((d, blk), jnp.float32) for _ in range(nr)]
    scratch += [pltpu.VMEM((blk, dp), jnp.float32) for _ in range(2)]
    scratch += [pltpu.SemaphoreType.DMA for _ in range(nr + 2)]

    @functools.partial(
        pl.kernel,
        mesh=mesh,
        out_type=jax.ShapeDtypeStruct((vocab_p, dp), jnp.float32),
        scratch_types=scratch,
        compiler_params=pltpu.CompilerParams(
            use_tc_tiling_on_sc=True, disable_bounds_checks=True,
            needs_layout_passes=False),
    )
    def padfmt_kernel(tt_hbm, out_hbm, *rest):
        sbufs = rest[:nr]
        dbufs = rest[nr:nr + 2]
        rsem = rest[nr + 2:2 * nr + 2]
        wsem = rest[2 * nr + 2:2 * nr + 4]
        wid = lax.axis_index("s") * nc + lax.axis_index("c")
        base = wid * slots

        def valid(s):
            return base + s < nblk

        def read_desc(s):
            return pltpu.make_async_copy(
                tt_hbm.at[:, pl.ds((base + s) * blk, blk)],
                sbufs[s % nr], rsem[s % nr])

        def write_desc(s):
            return pltpu.make_async_copy(
                dbufs[s % 2],
                out_hbm.at[pl.ds((base + s) * blk, blk)],
                wsem[s % 2])

        def when_valid(s, fn):
            pl.when(valid(s))(fn)

        lane = lax.iota(jnp.int32, 16)
        rot = [jnp.where(lane + k >= 16, lane + k - 16, lane + k)
               for k in range(16)]

        def transpose_blk(s):
            sbuf, dbuf = sbufs[s % nr], dbufs[s % 2]
            nfb = d // 16

            def body(i, carry):
                vb = i // nfb
                fb = i - vb * nfb
                rows = lane + fb * 16
                v0 = vb * 16
                cols = [rot[k] + v0 for k in range(16)]
                vs = [plsc.load_gather(sbuf, [rows, cols[k]])
                      for k in range(16)]
                for k in range(16):
                    plsc.store_scatter(dbuf, [cols[k], rows], vs[k])
                return carry

            lax.fori_loop(0, (blk // 16) * nfb, body, 0)

        for s in range(min(nr, slots)):
            when_valid(s, read_desc(s).start)
        for s in range(slots):
            when_valid(s, read_desc(s).wait)
            if s >= 2:
                when_valid(s - 2, write_desc(s - 2).wait)
            when_valid(s, lambda s=s: transpose_blk(s))
            when_valid(s, write_desc(s).start)
            if s + nr < slots:
                when_valid(s + nr, read_desc(s + nr).start)
        for s in range(max(0, slots - 2), slots):
            when_valid(s, write_desc(s).wait)

    return padfmt_kernel


@functools.lru_cache(maxsize=None)
def _build_gather(n: int, per_w: int, d: int, dp: int, nc: int, ns: int):
    chunk = 128
    nchunk = per_w // chunk
    ng = 5
    nt = 2
    super_steps = 2 * ng
    nsuper = (nchunk - ng) // super_steps
    assert nchunk == ng + nsuper * super_steps
    mesh = plsc.VectorSubcoreMesh(core_axis_name="c", subcore_axis_name="s")

    scratch = [pltpu.VMEM((per_w,), jnp.int32)]
    scratch += [pltpu.VMEM((chunk, dp), jnp.float32) for _ in range(ng)]
    scratch += [pltpu.VMEM((d, chunk), jnp.float32) for _ in range(nt)]
    scratch += [pltpu.SemaphoreType.DMA for _ in range(ng + nt)]

    @functools.partial(
        pl.kernel,
        mesh=mesh,
        out_type=jax.ShapeDtypeStruct((d, n), jnp.float32),
        scratch_types=scratch,
        compiler_params=pltpu.CompilerParams(
            use_tc_tiling_on_sc=True, disable_bounds_checks=True,
            needs_layout_passes=False),
    )
    def gather_kernel(x_hbm, table_hbm, out_hbm, idx_v, *rest):
        gbufs = rest[:ng]
        tbufs = rest[ng:ng + nt]
        gsem = rest[ng + nt:2 * ng + nt]
        wsem = rest[2 * ng + nt:2 * ng + 2 * nt]
        wid = lax.axis_index("s") * nc + lax.axis_index("c")
        base = wid * per_w
        pltpu.sync_copy(x_hbm.at[pl.ds(base, per_w)], idx_v)

        def valid(j):
            return base + j * chunk < n

        def gather_desc(j, gb):
            return pltpu.make_async_copy(
                table_hbm.at[idx_v.at[pl.ds(j * chunk, chunk)]],
                gbufs[gb], gsem[gb])

        def write_desc(j, tb):
            return pltpu.make_async_copy(
                tbufs[tb],
                out_hbm.at[:, pl.ds(base + j * chunk, chunk)],
                wsem[tb])

        def when_valid(j, fn):
            pl.when(valid(j))(fn)

        lane = lax.iota(jnp.int32, 16)
        rot = [jnp.where(lane + k >= 16, lane + k - 16, lane + k)
               for k in range(16)]

        def transpose_chunk(gb, tb):
            gbuf, tbuf = gbufs[gb], tbufs[tb]

            nfb = d // 16

            def body(i, carry):
                ob = i // nfb
                fb = i - ob * nfb
                rows = lane + ob * 16
                f0 = fb * 16
                cols = [rot[k] + f0 for k in range(16)]
                vs = [plsc.load_gather(gbuf, [rows, cols[k]])
                      for k in range(16)]
                for k in range(16):
                    plsc.store_scatter(tbuf, [cols[k], rows], vs[k])
                return carry

            lax.fori_loop(0, (chunk // 16) * nfb, body, 0)

        def step(j, gb, tb, static_head):
            if static_head:
                gather_desc(j, gb).wait()
                if j >= nt:
                    when_valid(j - nt, write_desc(j - nt, tb).wait)
            else:
                when_valid(j, gather_desc(j, gb).wait)
                when_valid(j - nt, write_desc(j - nt, tb).wait)
            when_valid(j, lambda: transpose_chunk(gb, tb))
            when_valid(j, write_desc(j, tb).start)
            jj = j + ng
            pl.when((jj < nchunk) & valid(jj))(gather_desc(jj, gb).start)

        for b in range(ng):
            gather_desc(b, b).start()
        for b in range(ng):
            step(b, b, b % nt, True)

        def super_group(sg, carry):
            j0 = ng + super_steps * sg
            for bb in range(super_steps):
                step(j0 + bb, (ng + bb) % ng, (ng + bb) % nt, False)
            return carry

        lax.fori_loop(0, nsuper, super_group, 0)

        for j in range(nchunk - nt, nchunk):
            when_valid(j, write_desc(j, j % nt).wait)

    return gather_kernel


def kernel(x, table):
    n = x.shape[0]
    vocab, d = table.shape
    dp = 128
    info = plsc.get_sparse_core_info()
    nc, ns = info.num_cores, info.num_subcores
    nw = nc * ns
    chunk = 128
    per_w = -(-n // (nw * chunk)) * chunk
    n_pad = per_w * nw

    xi = x.astype(jnp.int32)
    pad = jnp.arange(n_pad - n, dtype=jnp.int32) % vocab
    x_pad = jnp.concatenate([xi, pad])
    vocab_p = -(-vocab // 128) * 128
    table_p = _build_padfmt(vocab_p, d, dp, nc, ns)(table.T)
    out_t = _build_gather(n, per_w, d, dp, nc, ns)(x_pad, table_p)
    return out_t.T
